# Initial kernel scaffold; baseline (speedup 1.0000x reference)
import jax
import jax.numpy as jnp
from jax.experimental import pallas as pl


def kernel(img):
    raise NotImplementedError("write your pallas kernel here")


# speedup vs baseline: 82.5734x; 82.5734x over previous
import functools
import math

import jax
import jax.numpy as jnp
from jax import lax
from jax.experimental import pallas as pl
from jax.experimental.pallas import tpu as pltpu
from jax.experimental.pallas import tpu_sc as plsc

_N = 512
_CUTMIN = 2
_CUTMAX = 32
_ALPHA = 1.0

_NCH = 96
_BANDS = 16
_HALVES = 2
_BAND_ROWS = _N // _BANDS
_CH_PER_W = _NCH // _HALVES
_SRC_ROWS = 60
_BAND_PX = _BAND_ROWS * _N
_SRC_PX = _SRC_ROWS * _N
_VECS = _BAND_PX // 16


def _displacement_field():
    n = _N
    beta_sample = 0.5
    cut = int(beta_sample * (_CUTMAX + 1 - _CUTMIN) + _CUTMIN)
    c_ = cut + 1e-06
    log = math.log(c_)
    t1 = 1.0 / (math.pi * n ** 2 * log)
    t2 = 4.0 / (math.pi ** 3 * c_ ** 2 * log)
    t2 = max(t1, _ALPHA * t2)
    t = beta_sample * (t2 - t1) + t1

    x = jnp.linspace(0.0, 1.0, n, dtype=jnp.float32)
    k = jnp.arange(1, cut + 1, dtype=jnp.float32)
    i, j = jnp.meshgrid(k, k, indexing='ij')
    r = jnp.sqrt(i ** 2 + j ** 2)
    e = (r < cut + 0.5).astype(jnp.float32) / r
    s = jnp.sin(jnp.pi * x[:, None] * k[None, :])

    ku, kv = jax.random.split(jax.random.key(1))
    cu = jax.random.normal(ku, (cut, cut), dtype=jnp.float32) * e
    cv = jax.random.normal(kv, (cut, cut), dtype=jnp.float32) * e
    u = jnp.einsum('ij,xi,yj->yx', cu, s, s)
    v = jnp.einsum('ij,xi,yj->yx', cv, s, s)
    dx = (t ** 0.5) * u * n
    dy = (t ** 0.5) * v * n
    return dx, dy


def _gather_constants():
    n = _N
    dx, dy = _displacement_field()
    y, x = jnp.meshgrid(jnp.arange(n, dtype=jnp.float32),
                        jnp.arange(n, dtype=jnp.float32), indexing='ij')
    xn = jnp.clip(x - dx, 0.0, n - 1)
    yn = jnp.clip(y - dy, 0.0, n - 1)
    xf = jnp.minimum(jnp.floor(xn).astype(jnp.int32), n - 2)
    yf = jnp.minimum(jnp.floor(yn).astype(jnp.int32), n - 2)
    xv = xn - xf.astype(jnp.float32)
    yv = yn - yf.astype(jnp.float32)
    row = jnp.arange(n, dtype=jnp.int32)
    src_off = jnp.clip(_BAND_ROWS * (row // _BAND_ROWS) - 12, 0, n - _SRC_ROWS)
    i_tl = (yf - src_off[:, None]) * n + xf
    return i_tl.reshape(-1), xv.reshape(-1), yv.reshape(-1)


def _remap_body(img_hbm, it_hbm, xv_hbm, yv_hbm, out_hbm,
                it_v, xv_v, yv_v, src_a, src_b, out_v, sem_a, sem_b):
    j = lax.axis_index("s")
    h = lax.axis_index("c")
    r0 = j * _BAND_ROWS
    base_px = r0 * _N
    src_off = jnp.clip(r0 - 12, 0, _N - _SRC_ROWS)
    src_base = src_off * _N
    c0 = h * _CH_PER_W

    pltpu.sync_copy(it_hbm.at[pl.ds(base_px, _BAND_PX)], it_v)
    pltpu.sync_copy(xv_hbm.at[pl.ds(base_px, _BAND_PX)], xv_v)
    pltpu.sync_copy(yv_hbm.at[pl.ds(base_px, _BAND_PX)], yv_v)

    def issue(ci, buf, sem):
        pltpu.async_copy(img_hbm.at[c0 + ci, pl.ds(src_base, _SRC_PX)],
                         buf, sem)

    def wait(ci, buf, sem):
        pltpu.make_async_copy(img_hbm.at[c0 + ci, pl.ds(src_base, _SRC_PX)],
                              buf, sem).wait()

    def compute(ci, buf):
        @pl.loop(0, _VECS)
        def _inner(t):
            o = t * 16
            idx = it_v[pl.ds(o, 16)]
            wx = xv_v[pl.ds(o, 16)]
            wy = yv_v[pl.ds(o, 16)]
            a00 = plsc.load_gather(buf, [idx])
            a01 = plsc.load_gather(buf, [idx + 1])
            a10 = plsc.load_gather(buf, [idx + _N])
            a11 = plsc.load_gather(buf, [idx + (_N + 1)])
            top = a00 + wx * (a01 - a00)
            bot = a10 + wx * (a11 - a10)
            out_v[pl.ds(o, 16)] = top + wy * (bot - top)
        pltpu.sync_copy(out_v, out_hbm.at[c0 + ci, pl.ds(base_px, _BAND_PX)])

    issue(0, src_a, sem_a)
    issue(1, src_b, sem_b)

    @pl.loop(0, _CH_PER_W - 2, step=2)
    def _chan(ci):
        wait(ci, src_a, sem_a)
        compute(ci, src_a)
        issue(ci + 2, src_a, sem_a)
        wait(ci + 1, src_b, sem_b)
        compute(ci + 1, src_b)
        issue(ci + 3, src_b, sem_b)

    wait(_CH_PER_W - 2, src_a, sem_a)
    compute(_CH_PER_W - 2, src_a)
    wait(_CH_PER_W - 1, src_b, sem_b)
    compute(_CH_PER_W - 1, src_b)


@functools.partial(jax.jit, static_argnames=())
def _diffeo_remap(img2):
    i_tl, xv, yv = _gather_constants()
    mesh = plsc.VectorSubcoreMesh(core_axis_name="c", subcore_axis_name="s")
    fn = pl.kernel(
        _remap_body,
        out_type=jax.ShapeDtypeStruct((_NCH, _N * _N), jnp.float32),
        mesh=mesh,
        compiler_params=pltpu.CompilerParams(needs_layout_passes=False),
        scratch_types=[
            pltpu.VMEM((_BAND_PX,), jnp.int32),
            pltpu.VMEM((_BAND_PX,), jnp.float32),
            pltpu.VMEM((_BAND_PX,), jnp.float32),
            pltpu.VMEM((_SRC_PX,), jnp.float32),
            pltpu.VMEM((_SRC_PX,), jnp.float32),
            pltpu.VMEM((_BAND_PX,), jnp.float32),
            pltpu.SemaphoreType.DMA,
            pltpu.SemaphoreType.DMA,
        ],
    )
    return fn(img2, i_tl, xv, yv)


def kernel(img):
    init_shape = img.shape
    img2 = img.reshape(_NCH, _N * _N)
    out = _diffeo_remap(img2)
    return out.reshape(init_shape)
